# baseline (speedup 1.0000x reference)
import jax
import jax.numpy as jnp
from jax import lax
from jax.experimental import pallas as pl
from jax.experimental.pallas import tpu as pltpu
from jax.experimental.pallas import tpu_sc as plsc

_NC = 2
_NS = 16
_NW = _NC * _NS
_LANES = 16
_CHUNK = 32


def _emb_body(ids_hbm, tok_hbm, pos_hbm, out_hbm, idx_v, pos_v, buf, sem):
    n = ids_hbm.shape[0]
    hidden = tok_hbm.shape[1]
    max_pos = pos_hbm.shape[0]
    per_w = n // _NW
    nchunk = per_w // _CHUNK

    wid = lax.axis_index("s") * _NC + lax.axis_index("c")
    base = wid * per_w
    pltpu.sync_copy(ids_hbm.at[pl.ds(base, per_w)], idx_v)
    pltpu.sync_copy(pos_hbm, pos_v)

    @pl.loop(0, nchunk)
    def _chunk(c):
        pltpu.async_copy(
            tok_hbm.at[idx_v.at[pl.ds(c * _CHUNK, _CHUNK)]], buf, sem
        ).wait()
        off = lax.rem(c * _CHUNK, max_pos)

        @pl.loop(0, _CHUNK)
        def _row(r):
            for j in range(hidden // _LANES):
                sl = pl.ds(j * _LANES, _LANES)
                buf[r, sl] = buf[r, sl] + pos_v[off + r, sl]

        pltpu.sync_copy(buf, out_hbm.at[pl.ds(base + c * _CHUNK, _CHUNK)])


def kernel(input_ids, token_table, pos_table):
    b, s = input_ids.shape
    hidden = token_table.shape[1]
    max_pos = pos_table.shape[0]
    n = b * s
    ids_flat = input_ids.reshape(n).astype(jnp.int32)
    per_w = n // _NW

    mesh = plsc.VectorSubcoreMesh(core_axis_name="c", subcore_axis_name="s")
    run = pl.kernel(
        _emb_body,
        out_type=jax.ShapeDtypeStruct((n, hidden), jnp.float32),
        mesh=mesh,
        scratch_types=[
            pltpu.VMEM((per_w,), jnp.int32),
            pltpu.VMEM((max_pos, hidden), jnp.float32),
            pltpu.VMEM((_CHUNK, hidden), jnp.float32),
            pltpu.SemaphoreType.DMA,
        ],
    )
    out = run(ids_flat, token_table, pos_table)
    return out.reshape(b, s, hidden)


# speedup vs baseline: 1.7759x; 1.7759x over previous
import jax
import jax.numpy as jnp
from jax import lax
from jax.experimental import pallas as pl
from jax.experimental.pallas import tpu as pltpu
from jax.experimental.pallas import tpu_sc as plsc

_NC = 2
_NS = 16
_NW = _NC * _NS
_LANES = 16
_CHUNK = 32


def _emb_body(ids_hbm, tok_hbm, pos_hbm, out_hbm,
              idx_v, pos_v, buf0, buf1, buf2,
              sin0, sin1, sin2, sout0, sout1, sout2, spos):
    n = ids_hbm.shape[0]
    hidden = tok_hbm.shape[1]
    max_pos = pos_hbm.shape[0]
    per_w = n // _NW
    nchunk = per_w // _CHUNK
    bufs = (buf0, buf1, buf2)
    sins = (sin0, sin1, sin2)
    souts = (sout0, sout1, sout2)

    wid = lax.axis_index("s") * _NC + lax.axis_index("c")
    base = wid * per_w
    pltpu.sync_copy(ids_hbm.at[pl.ds(base, per_w)], idx_v)
    pos_cp = pltpu.async_copy(pos_hbm, pos_v, spos)

    def gather(x, p):
        pltpu.async_copy(
            tok_hbm.at[idx_v.at[pl.ds(x * _CHUNK, _CHUNK)]], bufs[p], sins[p])

    def wait_in(p):
        pltpu.make_async_copy(
            tok_hbm.at[pl.ds(0, _CHUNK)], bufs[p], sins[p]).wait()

    def put(x, p):
        pltpu.async_copy(
            bufs[p], out_hbm.at[pl.ds(base + x * _CHUNK, _CHUNK)], souts[p])

    def wait_out(p):
        pltpu.make_async_copy(
            bufs[p], out_hbm.at[pl.ds(0, _CHUNK)], souts[p]).wait()

    def vadd(x, p):
        off = lax.rem(x * _CHUNK, max_pos)

        @pl.loop(0, _CHUNK)
        def _row(r):
            for j in range(hidden // _LANES):
                sl = pl.ds(j * _LANES, _LANES)
                plsc.addupdate(bufs[p].at[r, sl], pos_v[off + r, sl])

    gather(0, 0)
    gather(1, 1)
    pos_cp.wait()

    @pl.loop(0, nchunk - 3, step=3)
    def _main(c):
        for k in range(3):
            x = c + k
            p = k
            q = (k + 2) % 3
            wait_in(p)
            vadd(x, p)
            put(x, p)
            if k == 0:
                @pl.when(c >= 1)
                def _():
                    wait_out(q)
            else:
                wait_out(q)
            if k == 2:
                @pl.when(c <= nchunk - 5)
                def _():
                    gather(x + 2, q)
            else:
                gather(x + 2, q)

    last = nchunk - 1
    wait_in(0)
    vadd(last, 0)
    wait_out(2)
    put(last, 0)
    wait_out(0)


def kernel(input_ids, token_table, pos_table):
    b, s = input_ids.shape
    hidden = token_table.shape[1]
    max_pos = pos_table.shape[0]
    n = b * s
    ids_flat = input_ids.reshape(n).astype(jnp.int32)
    per_w = n // _NW

    mesh = plsc.VectorSubcoreMesh(core_axis_name="c", subcore_axis_name="s")
    run = pl.kernel(
        _emb_body,
        out_type=jax.ShapeDtypeStruct((n, hidden), jnp.float32),
        mesh=mesh,
        scratch_types=[
            pltpu.VMEM((per_w,), jnp.int32),
            pltpu.VMEM((max_pos, hidden), jnp.float32),
            pltpu.VMEM((_CHUNK, hidden), jnp.float32),
            pltpu.VMEM((_CHUNK, hidden), jnp.float32),
            pltpu.VMEM((_CHUNK, hidden), jnp.float32),
            pltpu.SemaphoreType.DMA,
            pltpu.SemaphoreType.DMA,
            pltpu.SemaphoreType.DMA,
            pltpu.SemaphoreType.DMA,
            pltpu.SemaphoreType.DMA,
            pltpu.SemaphoreType.DMA,
            pltpu.SemaphoreType.DMA,
        ],
    )
    out = run(ids_flat, token_table, pos_table)
    return out.reshape(b, s, hidden)


# speedup vs baseline: 3.5535x; 2.0010x over previous
import jax
import jax.numpy as jnp
from jax import lax
from jax.experimental import pallas as pl
from jax.experimental.pallas import tpu as pltpu
from jax.experimental.pallas import tpu_sc as plsc

_NC = 2
_NS = 16
_NW = _NC * _NS
_LANES = 16
_CHUNK = 32


def _emb_body(ids_hbm, tok_hbm, pos_hbm, out_hbm,
              idx_v, pos_v, buf0, buf1, buf2,
              sin0, sin1, sin2, sout0, sout1, sout2, spos):
    n = ids_hbm.shape[0]
    hidden = tok_hbm.shape[1]
    max_pos = pos_hbm.shape[0]
    per_w = n // _NW
    nchunk = per_w // _CHUNK
    bufs = (buf0, buf1, buf2)
    sins = (sin0, sin1, sin2)
    souts = (sout0, sout1, sout2)

    wid = lax.axis_index("s") * _NC + lax.axis_index("c")
    base = wid * per_w
    pltpu.sync_copy(ids_hbm.at[pl.ds(base, per_w)], idx_v)
    pos_cp = pltpu.async_copy(pos_hbm, pos_v, spos)

    def gather(x, p):
        pltpu.async_copy(
            tok_hbm.at[idx_v.at[pl.ds(x * _CHUNK, _CHUNK)]], bufs[p], sins[p])

    def wait_in(p):
        pltpu.make_async_copy(
            tok_hbm.at[pl.ds(0, _CHUNK)], bufs[p], sins[p]).wait()

    def put(x, p):
        pltpu.async_copy(
            bufs[p], out_hbm.at[pl.ds(base + x * _CHUNK, _CHUNK)], souts[p])

    def wait_out(p):
        pltpu.make_async_copy(
            bufs[p], out_hbm.at[pl.ds(0, _CHUNK)], souts[p]).wait()

    def vadd(x, p):
        off = lax.rem(x * _CHUNK, max_pos)

        @plsc.parallel_loop(0, _CHUNK, unroll=2)
        def _row(r):
            for j in range(hidden // _LANES):
                sl = pl.ds(j * _LANES, _LANES)
                plsc.addupdate(bufs[p].at[r, sl], pos_v[off + r, sl])

    gather(0, 0)
    gather(1, 1)
    pos_cp.wait()

    @pl.loop(0, nchunk - 3, step=3)
    def _main(c):
        for k in range(3):
            x = c + k
            p = k
            q = (k + 2) % 3
            wait_in(p)
            vadd(x, p)
            put(x, p)
            if k == 0:
                @pl.when(c >= 1)
                def _():
                    wait_out(q)
            else:
                wait_out(q)
            if k == 2:
                @pl.when(c <= nchunk - 5)
                def _():
                    gather(x + 2, q)
            else:
                gather(x + 2, q)

    last = nchunk - 1
    wait_in(0)
    vadd(last, 0)
    wait_out(2)
    put(last, 0)
    wait_out(0)


def kernel(input_ids, token_table, pos_table):
    b, s = input_ids.shape
    hidden = token_table.shape[1]
    max_pos = pos_table.shape[0]
    n = b * s
    ids_flat = input_ids.reshape(n).astype(jnp.int32)
    per_w = n // _NW

    mesh = plsc.VectorSubcoreMesh(core_axis_name="c", subcore_axis_name="s")
    run = pl.kernel(
        _emb_body,
        out_type=jax.ShapeDtypeStruct((n, hidden), jnp.float32),
        mesh=mesh,
        scratch_types=[
            pltpu.VMEM((per_w,), jnp.int32),
            pltpu.VMEM((max_pos, hidden), jnp.float32),
            pltpu.VMEM((_CHUNK, hidden), jnp.float32),
            pltpu.VMEM((_CHUNK, hidden), jnp.float32),
            pltpu.VMEM((_CHUNK, hidden), jnp.float32),
            pltpu.SemaphoreType.DMA,
            pltpu.SemaphoreType.DMA,
            pltpu.SemaphoreType.DMA,
            pltpu.SemaphoreType.DMA,
            pltpu.SemaphoreType.DMA,
            pltpu.SemaphoreType.DMA,
            pltpu.SemaphoreType.DMA,
        ],
    )
    out = run(ids_flat, token_table, pos_table)
    return out.reshape(b, s, hidden)
